# baseline (speedup 1.0000x reference)
import functools

import jax
import jax.numpy as jnp
from jax import lax
from jax.experimental import pallas as pl
from jax.experimental.pallas import tpu as pltpu
from jax.experimental.pallas import tpu_sc as plsc

_T, _D, _E, _C = 8192, 768, 64, 320
_DFF, _DFFS = 256, 1024
_XE_ROWS = (_E + 1) * _C
_DUMP = _E * _C
_NC, _NS = 2, 16
_NW = _NC * _NS
_TPW = _T // _NW
_CHUNK = 64
_RB = 2048
_TB = 1024


def _router_body(x_ref, wr_ref, idx_ref, gates_ref, xp_ref, carry_ref):
    i = pl.program_id(0)

    @pl.when(i == 0)
    def _init():
        carry_ref[...] = jnp.zeros_like(carry_ref)

    x = x_ref[...]
    w = wr_ref[...]

    logits = lax.dot_general(x, w, (((1,), (1,)), ((), ())))
    m1 = jnp.max(logits, axis=1, keepdims=True)
    lane = lax.broadcasted_iota(jnp.int32, (_RB, _E), 1)
    i1 = jnp.min(jnp.where(logits == m1, lane, _E), axis=1, keepdims=True)
    masked = jnp.where(lane == i1, -jnp.inf, logits)
    m2 = jnp.max(masked, axis=1, keepdims=True)
    g0 = jax.nn.sigmoid(m1 - m2)
    g1 = jax.nn.sigmoid(m2 - m1)
    gates_ref[...] = jnp.concatenate([g0, g1], axis=1)

    lt = lax.dot_general(w, x, (((1,), (1,)), ((), ())))
    sub = lax.broadcasted_iota(jnp.int32, (_E, _RB), 0)
    m1t = jnp.max(lt, axis=0, keepdims=True)
    i1t = jnp.min(jnp.where(lt == m1t, sub, _E), axis=0, keepdims=True)
    oh0 = (sub == i1t).astype(jnp.float32)
    maskedt = jnp.where(oh0 > 0, -jnp.inf, lt)
    m2t = jnp.max(maskedt, axis=0, keepdims=True)
    i2t = jnp.min(jnp.where(maskedt == m2t, sub, _E), axis=0, keepdims=True)
    oh1 = (sub == i2t).astype(jnp.float32)

    contrib = oh0 + oh1
    r = lax.broadcasted_iota(jnp.int32, (_RB, _RB), 0)
    c = lax.broadcasted_iota(jnp.int32, (_RB, _RB), 1)
    striu = (r < c).astype(jnp.float32)
    excl = lax.dot_general(contrib, striu, (((1,), (0,)), ((), ())))
    excl = excl + carry_ref[...]
    carry_ref[...] = carry_ref[...] + jnp.sum(contrib, axis=1, keepdims=True)

    p0 = jnp.sum(excl * oh0, axis=0, keepdims=True)
    p1 = jnp.sum(excl * oh1, axis=0, keepdims=True)
    slot0 = i1t.astype(jnp.float32) * _C + p0
    slot1 = i2t.astype(jnp.float32) * _C + p1
    s0 = jnp.where(p0 < _C, slot0, float(_DUMP)).astype(jnp.int32)
    s1 = jnp.where(p1 < _C, slot1, float(_DUMP)).astype(jnp.int32)
    idx_ref[...] = jnp.concatenate([s0, s1], axis=0)

    xb = x.astype(_BF)
    lo = lax.bitcast_convert_type(xb[:, :_D // 2], jnp.uint16)
    hi = lax.bitcast_convert_type(xb[:, _D // 2:], jnp.uint16)
    xp_ref[...] = lo.astype(jnp.uint32) | (hi.astype(jnp.uint32) << 16)


_router = pl.pallas_call(
    _router_body,
    grid=(_T // _RB,),
    in_specs=[
        pl.BlockSpec((_RB, _D), lambda i: (i, 0)),
        pl.BlockSpec((_E, _D), lambda i: (0, 0)),
    ],
    out_specs=[
        pl.BlockSpec((2, _RB), lambda i: (0, i)),
        pl.BlockSpec((_RB, 2), lambda i: (i, 0)),
        pl.BlockSpec((_RB, _D // 2), lambda i: (i, 0)),
    ],
    out_shape=[
        jax.ShapeDtypeStruct((2, _T), jnp.int32),
        jax.ShapeDtypeStruct((_T, 2), jnp.float32),
        jax.ShapeDtypeStruct((_T, _D // 2), jnp.uint32),
    ],
    scratch_shapes=[pltpu.VMEM((_E, 1), jnp.float32)],
)


@functools.cache
def _make_dispatch():
    mesh = plsc.VectorSubcoreMesh(
        core_axis_name="c", subcore_axis_name="s",
        num_cores=_NC, num_subcores=_NS)

    @functools.partial(
        pl.kernel,
        out_type=jax.ShapeDtypeStruct((_XE_ROWS, _D // 2), jnp.uint32),
        mesh=mesh,
        scratch_types=[
            pltpu.VMEM((_CHUNK, _D // 2), jnp.uint32),
            pltpu.VMEM((_CHUNK, _D // 2), jnp.uint32),
            pltpu.VMEM((_CHUNK,), jnp.int32),
            pltpu.VMEM((_CHUNK,), jnp.int32),
            pltpu.VMEM((_CHUNK,), jnp.int32),
            pltpu.VMEM((_CHUNK,), jnp.int32),
            pltpu.SemaphoreType.DMA,
            pltpu.SemaphoreType.DMA,
        ],
    )
    def dispatch(xp_hbm, idx_hbm, xe_hbm,
                 xv0, xv1, d0v0, d0v1, d1v0, d1v1, lsem, ssem):
        wid = lax.axis_index("s") * _NC + lax.axis_index("c")
        base = wid * _TPW
        bufs = ((xv0, d0v0, d1v0), (xv1, d0v1, d1v1))
        nch = _TPW // _CHUNK

        def start_loads(ci, xvb, d0b, d1b):
            off = base + ci * _CHUNK
            return (
                pltpu.async_copy(xp_hbm.at[pl.ds(off, _CHUNK)], xvb, lsem),
                pltpu.async_copy(idx_hbm.at[0, pl.ds(off, _CHUNK)], d0b, lsem),
                pltpu.async_copy(idx_hbm.at[1, pl.ds(off, _CHUNK)], d1b, lsem),
            )

        pend = start_loads(0, *bufs[0])
        prev = None
        for ci in range(nch):
            for d in pend:
                d.wait()
            if prev is not None:
                for d in prev:
                    d.wait()
            xvb, d0b, d1b = bufs[ci % 2]
            scat = (pltpu.async_copy(xvb, xe_hbm.at[d0b], ssem),
                    pltpu.async_copy(xvb, xe_hbm.at[d1b], ssem))
            if ci + 1 < nch:
                pend = start_loads(ci + 1, *bufs[(ci + 1) % 2])
            prev = scat
        for d in prev:
            d.wait()

    return dispatch


_DN = (((1,), (1,)), ((), ()))
_F32 = jnp.float32
_BF = jnp.bfloat16


def _unpack_bf16_pair_bf(p):
    lo = lax.bitcast_convert_type((p & 0xFFFF).astype(jnp.uint16), _BF)
    hi = lax.bitcast_convert_type((p >> 16).astype(jnp.uint16), _BF)
    return jnp.concatenate([lo, hi], axis=1)


def _expert_body(xe_ref, w1_ref, w3_ref, w2_ref, yg_ref):
    e = pl.program_id(0)

    @pl.when(e < _E)
    def _compute():
        xe = _unpack_bf16_pair_bf(xe_ref[...])
        a = lax.dot_general(xe, w1_ref[0].astype(_BF), _DN,
                            preferred_element_type=_F32)
        b = lax.dot_general(xe, w3_ref[0].astype(_BF), _DN,
                            preferred_element_type=_F32)
        h = (a * jax.nn.sigmoid(a)) * b
        y = lax.dot_general(h.astype(_BF), w2_ref[0].astype(_BF), _DN,
                            preferred_element_type=_F32)
        yb = y.astype(_BF)
        lo = lax.bitcast_convert_type(yb[:, :_D // 2], jnp.uint16)
        hi = lax.bitcast_convert_type(yb[:, _D // 2:], jnp.uint16)
        yg_ref[...] = lo.astype(jnp.uint32) | (hi.astype(jnp.uint32) << 16)

    @pl.when(e == _E)
    def _zero():
        yg_ref[...] = jnp.zeros_like(yg_ref)


_experts = pl.pallas_call(
    _expert_body,
    grid=(_E + 1,),
    in_specs=[
        pl.BlockSpec((_C, _D // 2), lambda e: (e, 0)),
        pl.BlockSpec((1, _DFF, _D), lambda e: (jnp.minimum(e, _E - 1), 0, 0)),
        pl.BlockSpec((1, _DFF, _D), lambda e: (jnp.minimum(e, _E - 1), 0, 0)),
        pl.BlockSpec((1, _D, _DFF), lambda e: (jnp.minimum(e, _E - 1), 0, 0)),
    ],
    out_specs=pl.BlockSpec((_C, _D // 2), lambda e: (e, 0)),
    out_shape=jax.ShapeDtypeStruct((_XE_ROWS, _D // 2), jnp.uint32),
)


@functools.cache
def _make_combine_gather():
    mesh = plsc.VectorSubcoreMesh(
        core_axis_name="c", subcore_axis_name="s",
        num_cores=_NC, num_subcores=_NS)

    @functools.partial(
        pl.kernel,
        out_type=(
            jax.ShapeDtypeStruct((_T, _D // 2), jnp.uint32),
            jax.ShapeDtypeStruct((_T, _D // 2), jnp.uint32),
        ),
        mesh=mesh,
        scratch_types=[
            pltpu.VMEM((_CHUNK, _D // 2), jnp.uint32),
            pltpu.VMEM((_CHUNK,), jnp.int32),
            pltpu.SemaphoreType.DMA,
        ],
    )
    def combine(yg_hbm, idx_hbm, y0_hbm, y1_hbm, rows, idxv, sem):
        wid = lax.axis_index("s") * _NC + lax.axis_index("c")
        base = wid * _TPW

        def body(ci, carry):
            off = base + ci * _CHUNK
            pltpu.sync_copy(idx_hbm.at[0, pl.ds(off, _CHUNK)], idxv)
            pltpu.async_copy(yg_hbm.at[idxv], rows, sem).wait()
            pltpu.sync_copy(rows, y0_hbm.at[pl.ds(off, _CHUNK)])
            pltpu.sync_copy(idx_hbm.at[1, pl.ds(off, _CHUNK)], idxv)
            pltpu.async_copy(yg_hbm.at[idxv], rows, sem).wait()
            pltpu.sync_copy(rows, y1_hbm.at[pl.ds(off, _CHUNK)])
            return carry

        lax.fori_loop(0, _TPW // _CHUNK, body, 0)

    return combine


def _shared_body(xp_ref, ws1_ref, ws3_ref, ws2_ref, sh_ref):
    x = _unpack_bf16_pair_bf(xp_ref[...])
    a = lax.dot_general(x, ws1_ref[...].astype(_BF), _DN,
                        preferred_element_type=_F32)
    b = lax.dot_general(x, ws3_ref[...].astype(_BF), _DN,
                        preferred_element_type=_F32)
    hs = (a * jax.nn.sigmoid(a)) * b
    shared = lax.dot_general(hs.astype(_BF), ws2_ref[...].astype(_BF), _DN,
                             preferred_element_type=_F32)
    sh_ref[...] = shared.astype(_BF)


_STB = 1024
_NBA = 6
_SBA = _NBA * _TB // _STB


def _make_shared(nblocks, block_off):
    return pl.pallas_call(
        _shared_body,
        grid=(nblocks,),
        in_specs=[
            pl.BlockSpec((_STB, _D // 2), lambda i: (i + block_off, 0)),
            pl.BlockSpec((_DFFS, _D), lambda i: (0, 0)),
            pl.BlockSpec((_DFFS, _D), lambda i: (0, 0)),
            pl.BlockSpec((_D, _DFFS), lambda i: (0, 0)),
        ],
        out_specs=pl.BlockSpec((_STB, _D), lambda i: (i, 0)),
        out_shape=jax.ShapeDtypeStruct((nblocks * _STB, _D), jnp.bfloat16),
    )


_shared_a = _make_shared(_SBA, 0)
_shared_b = _make_shared(_T // _STB - _SBA, _SBA)


def _unpack_bf16_pair(p):
    lo = lax.bitcast_convert_type((p & 0xFFFF).astype(jnp.uint16), _BF)
    hi = lax.bitcast_convert_type((p >> 16).astype(jnp.uint16), _BF)
    return jnp.concatenate([lo, hi], axis=1).astype(_F32)


def _final_body(sha_ref, shb_ref, y0_ref, y1_ref, gates_ref, out_ref):
    i = pl.program_id(0)
    y0 = _unpack_bf16_pair(y0_ref[...])
    y1 = _unpack_bf16_pair(y1_ref[...])
    g0 = gates_ref[:, 0:1]
    g1 = gates_ref[:, 1:2]
    sh = jnp.where(i < _NBA, sha_ref[...], shb_ref[...]).astype(_F32)
    out_ref[...] = sh + g0 * y0 + g1 * y1


_final = pl.pallas_call(
    _final_body,
    grid=(_T // _TB,),
    in_specs=[
        pl.BlockSpec((_TB, _D), lambda i: (jnp.minimum(i, _NBA - 1), 0)),
        pl.BlockSpec((_TB, _D), lambda i: (jnp.maximum(i - _NBA, 0), 0)),
        pl.BlockSpec((_TB, _D // 2), lambda i: (i, 0)),
        pl.BlockSpec((_TB, _D // 2), lambda i: (i, 0)),
        pl.BlockSpec((_TB, 2), lambda i: (i, 0)),
    ],
    out_specs=pl.BlockSpec((_TB, _D), lambda i: (i, 0)),
    out_shape=jax.ShapeDtypeStruct((_T, _D), jnp.float32),
)


def kernel(x, W_router, w1, w2, w3, ws1, ws2, ws3):
    idx_t, gates, xp = _router(x, W_router)
    xe = _make_dispatch()(xp, idx_t)
    sh_a = _shared_a(xp, ws1, ws3, ws2)
    xe, sh_a = lax.optimization_barrier((xe, sh_a))
    yg = _experts(xe, w1, w3, w2)
    y0, y1 = _make_combine_gather()(yg, idx_t)
    xp_b, _ = lax.optimization_barrier((xp, yg))
    sh_b = _shared_b(xp_b, ws1, ws3, ws2)
    return _final(sh_a, sh_b, y0, y1, gates)


# speedup vs baseline: 1.0494x; 1.0494x over previous
import functools

import jax
import jax.numpy as jnp
from jax import lax
from jax.experimental import pallas as pl
from jax.experimental.pallas import tpu as pltpu
from jax.experimental.pallas import tpu_sc as plsc

_T, _D, _E, _C = 8192, 768, 64, 320
_DFF, _DFFS = 256, 1024
_XE_ROWS = (_E + 1) * _C
_DUMP = _E * _C
_NC, _NS = 2, 16
_NW = _NC * _NS
_TPW = _T // _NW
_CHUNK = 64
_RB = 2048
_TB = 512


def _router_body(x_ref, wr_ref, idx_ref, gates_ref, xp_ref, carry_ref):
    i = pl.program_id(0)

    @pl.when(i == 0)
    def _init():
        carry_ref[...] = jnp.zeros_like(carry_ref)

    x = x_ref[...]
    w = wr_ref[...]

    logits = lax.dot_general(x, w, (((1,), (1,)), ((), ())))
    m1 = jnp.max(logits, axis=1, keepdims=True)
    lane = lax.broadcasted_iota(jnp.int32, (_RB, _E), 1)
    i1 = jnp.min(jnp.where(logits == m1, lane, _E), axis=1, keepdims=True)
    masked = jnp.where(lane == i1, -jnp.inf, logits)
    m2 = jnp.max(masked, axis=1, keepdims=True)
    g0 = jax.nn.sigmoid(m1 - m2)
    g1 = jax.nn.sigmoid(m2 - m1)
    gates_ref[...] = jnp.concatenate([g0, g1], axis=1)

    lt = lax.dot_general(w, x, (((1,), (1,)), ((), ())))
    sub = lax.broadcasted_iota(jnp.int32, (_E, _RB), 0)
    m1t = jnp.max(lt, axis=0, keepdims=True)
    i1t = jnp.min(jnp.where(lt == m1t, sub, _E), axis=0, keepdims=True)
    oh0 = (sub == i1t).astype(jnp.float32)
    maskedt = jnp.where(oh0 > 0, -jnp.inf, lt)
    m2t = jnp.max(maskedt, axis=0, keepdims=True)
    i2t = jnp.min(jnp.where(maskedt == m2t, sub, _E), axis=0, keepdims=True)
    oh1 = (sub == i2t).astype(jnp.float32)

    contrib = oh0 + oh1
    r = lax.broadcasted_iota(jnp.int32, (_RB, _RB), 0)
    c = lax.broadcasted_iota(jnp.int32, (_RB, _RB), 1)
    striu = (r < c).astype(jnp.float32)
    excl = lax.dot_general(contrib, striu, (((1,), (0,)), ((), ())))
    excl = excl + carry_ref[...]
    carry_ref[...] = carry_ref[...] + jnp.sum(contrib, axis=1, keepdims=True)

    p0 = jnp.sum(excl * oh0, axis=0, keepdims=True)
    p1 = jnp.sum(excl * oh1, axis=0, keepdims=True)
    slot0 = i1t.astype(jnp.float32) * _C + p0
    slot1 = i2t.astype(jnp.float32) * _C + p1
    s0 = jnp.where(p0 < _C, slot0, float(_DUMP)).astype(jnp.int32)
    s1 = jnp.where(p1 < _C, slot1, float(_DUMP)).astype(jnp.int32)
    idx_ref[...] = jnp.concatenate([s0, s1], axis=0)

    xb = x.astype(_BF)
    lo = lax.bitcast_convert_type(xb[:, :_D // 2], jnp.uint16)
    hi = lax.bitcast_convert_type(xb[:, _D // 2:], jnp.uint16)
    xp_ref[...] = lo.astype(jnp.uint32) | (hi.astype(jnp.uint32) << 16)


_router = pl.pallas_call(
    _router_body,
    grid=(_T // _RB,),
    in_specs=[
        pl.BlockSpec((_RB, _D), lambda i: (i, 0)),
        pl.BlockSpec((_E, _D), lambda i: (0, 0)),
    ],
    out_specs=[
        pl.BlockSpec((2, _RB), lambda i: (0, i)),
        pl.BlockSpec((_RB, 2), lambda i: (i, 0)),
        pl.BlockSpec((_RB, _D // 2), lambda i: (i, 0)),
    ],
    out_shape=[
        jax.ShapeDtypeStruct((2, _T), jnp.int32),
        jax.ShapeDtypeStruct((_T, 2), jnp.float32),
        jax.ShapeDtypeStruct((_T, _D // 2), jnp.uint32),
    ],
    scratch_shapes=[pltpu.VMEM((_E, 1), jnp.float32)],
)


@functools.cache
def _make_dispatch():
    mesh = plsc.VectorSubcoreMesh(
        core_axis_name="c", subcore_axis_name="s",
        num_cores=_NC, num_subcores=_NS)

    @functools.partial(
        pl.kernel,
        out_type=jax.ShapeDtypeStruct((_XE_ROWS, _D // 2), jnp.uint32),
        mesh=mesh,
        scratch_types=[
            pltpu.VMEM((_CHUNK, _D // 2), jnp.uint32),
            pltpu.VMEM((_CHUNK, _D // 2), jnp.uint32),
            pltpu.VMEM((_CHUNK,), jnp.int32),
            pltpu.VMEM((_CHUNK,), jnp.int32),
            pltpu.VMEM((_CHUNK,), jnp.int32),
            pltpu.VMEM((_CHUNK,), jnp.int32),
            pltpu.SemaphoreType.DMA,
            pltpu.SemaphoreType.DMA,
        ],
    )
    def dispatch(xp_hbm, idx_hbm, xe_hbm,
                 xv0, xv1, d0v0, d0v1, d1v0, d1v1, lsem, ssem):
        wid = lax.axis_index("s") * _NC + lax.axis_index("c")
        base = wid * _TPW
        bufs = ((xv0, d0v0, d1v0), (xv1, d0v1, d1v1))
        nch = _TPW // _CHUNK

        def start_loads(ci, xvb, d0b, d1b):
            off = base + ci * _CHUNK
            return (
                pltpu.async_copy(xp_hbm.at[pl.ds(off, _CHUNK)], xvb, lsem),
                pltpu.async_copy(idx_hbm.at[0, pl.ds(off, _CHUNK)], d0b, lsem),
                pltpu.async_copy(idx_hbm.at[1, pl.ds(off, _CHUNK)], d1b, lsem),
            )

        pend = start_loads(0, *bufs[0])
        prev = None
        for ci in range(nch):
            for d in pend:
                d.wait()
            if prev is not None:
                for d in prev:
                    d.wait()
            xvb, d0b, d1b = bufs[ci % 2]
            scat = (pltpu.async_copy(xvb, xe_hbm.at[d0b], ssem),
                    pltpu.async_copy(xvb, xe_hbm.at[d1b], ssem))
            if ci + 1 < nch:
                pend = start_loads(ci + 1, *bufs[(ci + 1) % 2])
            prev = scat
        for d in prev:
            d.wait()

    return dispatch


_DN = (((1,), (1,)), ((), ()))
_F32 = jnp.float32
_BF = jnp.bfloat16


def _unpack_bf16_pair_bf(p):
    lo = lax.bitcast_convert_type((p & 0xFFFF).astype(jnp.uint16), _BF)
    hi = lax.bitcast_convert_type((p >> 16).astype(jnp.uint16), _BF)
    return jnp.concatenate([lo, hi], axis=1)


def _expert_body(xe_ref, w1_ref, w3_ref, w2_ref, yg_ref):
    e = pl.program_id(0)

    @pl.when(e < _E)
    def _compute():
        xe = _unpack_bf16_pair_bf(xe_ref[...])
        a = lax.dot_general(xe, w1_ref[0].astype(_BF), _DN,
                            preferred_element_type=_F32)
        b = lax.dot_general(xe, w3_ref[0].astype(_BF), _DN,
                            preferred_element_type=_F32)
        h = (a * jax.nn.sigmoid(a)) * b
        y = lax.dot_general(h.astype(_BF), w2_ref[0].astype(_BF), _DN,
                            preferred_element_type=_F32)
        yb = y.astype(_BF)
        lo = lax.bitcast_convert_type(yb[:, :_D // 2], jnp.uint16)
        hi = lax.bitcast_convert_type(yb[:, _D // 2:], jnp.uint16)
        yg_ref[...] = lo.astype(jnp.uint32) | (hi.astype(jnp.uint32) << 16)

    @pl.when(e == _E)
    def _zero():
        yg_ref[...] = jnp.zeros_like(yg_ref)


_experts = pl.pallas_call(
    _expert_body,
    grid=(_E + 1,),
    in_specs=[
        pl.BlockSpec((_C, _D // 2), lambda e: (e, 0)),
        pl.BlockSpec((1, _DFF, _D), lambda e: (jnp.minimum(e, _E - 1), 0, 0)),
        pl.BlockSpec((1, _DFF, _D), lambda e: (jnp.minimum(e, _E - 1), 0, 0)),
        pl.BlockSpec((1, _D, _DFF), lambda e: (jnp.minimum(e, _E - 1), 0, 0)),
    ],
    out_specs=pl.BlockSpec((_C, _D // 2), lambda e: (e, 0)),
    out_shape=jax.ShapeDtypeStruct((_XE_ROWS, _D // 2), jnp.uint32),
)


@functools.cache
def _make_combine_gather():
    mesh = plsc.VectorSubcoreMesh(
        core_axis_name="c", subcore_axis_name="s",
        num_cores=_NC, num_subcores=_NS)

    @functools.partial(
        pl.kernel,
        out_type=(
            jax.ShapeDtypeStruct((_T, _D // 2), jnp.uint32),
            jax.ShapeDtypeStruct((_T, _D // 2), jnp.uint32),
        ),
        mesh=mesh,
        scratch_types=[
            pltpu.VMEM((_CHUNK, _D // 2), jnp.uint32),
            pltpu.VMEM((_CHUNK,), jnp.int32),
            pltpu.SemaphoreType.DMA,
        ],
    )
    def combine(yg_hbm, idx_hbm, y0_hbm, y1_hbm, rows, idxv, sem):
        wid = lax.axis_index("s") * _NC + lax.axis_index("c")
        base = wid * _TPW

        def body(ci, carry):
            off = base + ci * _CHUNK
            pltpu.sync_copy(idx_hbm.at[0, pl.ds(off, _CHUNK)], idxv)
            pltpu.async_copy(yg_hbm.at[idxv], rows, sem).wait()
            pltpu.sync_copy(rows, y0_hbm.at[pl.ds(off, _CHUNK)])
            pltpu.sync_copy(idx_hbm.at[1, pl.ds(off, _CHUNK)], idxv)
            pltpu.async_copy(yg_hbm.at[idxv], rows, sem).wait()
            pltpu.sync_copy(rows, y1_hbm.at[pl.ds(off, _CHUNK)])
            return carry

        lax.fori_loop(0, _TPW // _CHUNK, body, 0)

    return combine


def _shared_body(xp_ref, ws1_ref, ws3_ref, ws2_ref, sh_ref):
    x = _unpack_bf16_pair_bf(xp_ref[...])
    a = lax.dot_general(x, ws1_ref[...].astype(_BF), _DN,
                        preferred_element_type=_F32)
    b = lax.dot_general(x, ws3_ref[...].astype(_BF), _DN,
                        preferred_element_type=_F32)
    hs = (a * jax.nn.sigmoid(a)) * b
    shared = lax.dot_general(hs.astype(_BF), ws2_ref[...].astype(_BF), _DN,
                             preferred_element_type=_F32)
    sh_ref[...] = shared.astype(_BF)


_STB = 1024
_NBA = 6
_SBA = _NBA * _TB // _STB


def _make_shared(nblocks, block_off):
    return pl.pallas_call(
        _shared_body,
        grid=(nblocks,),
        in_specs=[
            pl.BlockSpec((_STB, _D // 2), lambda i: (i + block_off, 0)),
            pl.BlockSpec((_DFFS, _D), lambda i: (0, 0)),
            pl.BlockSpec((_DFFS, _D), lambda i: (0, 0)),
            pl.BlockSpec((_D, _DFFS), lambda i: (0, 0)),
        ],
        out_specs=pl.BlockSpec((_STB, _D), lambda i: (i, 0)),
        out_shape=jax.ShapeDtypeStruct((nblocks * _STB, _D), jnp.bfloat16),
    )


_shared_a = _make_shared(_SBA, 0)
_shared_b = _make_shared(_T // _STB - _SBA, _SBA)


def _unpack_bf16_pair(p):
    lo = lax.bitcast_convert_type((p & 0xFFFF).astype(jnp.uint16), _BF)
    hi = lax.bitcast_convert_type((p >> 16).astype(jnp.uint16), _BF)
    return jnp.concatenate([lo, hi], axis=1).astype(_F32)


def _final_body(sha_ref, shb_ref, y0_ref, y1_ref, gates_ref, out_ref):
    i = pl.program_id(0)
    y0 = _unpack_bf16_pair(y0_ref[...])
    y1 = _unpack_bf16_pair(y1_ref[...])
    g0 = gates_ref[:, 0:1]
    g1 = gates_ref[:, 1:2]
    sh = jnp.where(i < _NBA, sha_ref[...], shb_ref[...]).astype(_F32)
    out_ref[...] = sh + g0 * y0 + g1 * y1


_final = pl.pallas_call(
    _final_body,
    grid=(_T // _TB,),
    in_specs=[
        pl.BlockSpec((_TB, _D), lambda i: (jnp.minimum(i, _NBA - 1), 0)),
        pl.BlockSpec((_TB, _D), lambda i: (jnp.maximum(i - _NBA, 0), 0)),
        pl.BlockSpec((_TB, _D // 2), lambda i: (i, 0)),
        pl.BlockSpec((_TB, _D // 2), lambda i: (i, 0)),
        pl.BlockSpec((_TB, 2), lambda i: (i, 0)),
    ],
    out_specs=pl.BlockSpec((_TB, _D), lambda i: (i, 0)),
    out_shape=jax.ShapeDtypeStruct((_T, _D), jnp.float32),
)


def kernel(x, W_router, w1, w2, w3, ws1, ws2, ws3):
    idx_t, gates, xp = _router(x, W_router)
    xe = _make_dispatch()(xp, idx_t)
    sh_a = _shared_a(xp, ws1, ws3, ws2)
    xe, sh_a = lax.optimization_barrier((xe, sh_a))
    yg = _experts(xe, w1, w3, w2)
    y0, y1 = _make_combine_gather()(yg, idx_t)
    xp_b, _ = lax.optimization_barrier((xp, yg))
    sh_b = _shared_b(xp_b, ws1, ws3, ws2)
    return _final(sh_a, sh_b, y0, y1, gates)
